# Initial kernel scaffold; baseline (speedup 1.0000x reference)
import jax
import jax.numpy as jnp
from jax.experimental import pallas as pl


def kernel(emb_ent, emb_rel, Wn0, Wl0, Wn1, Wl1, edge_index, edge_type):
    raise NotImplementedError("write your pallas kernel here")


# speedup vs baseline: 4.8271x; 4.8271x over previous
import functools

import jax
import jax.numpy as jnp
from jax import lax
from jax.experimental import pallas as pl
from jax.experimental.pallas import tpu as pltpu
from jax.experimental.pallas import tpu_sc as plsc

N = 10000
E = 320000
D = 128
H = D // 2
NREL = 460
RRELU_SLOPE = (1.0 / 8.0 + 1.0 / 3.0) / 2.0

NP = 10240
NC = 2
NS = 16
NW = NC * NS
EW = E // NW
B = 80
CHUNKS = EW // B
RPT = NP // NS
ZR = 128


def _fill_f32_1d(ref, nwords, value):
    v = jnp.full((16,), value, jnp.float32)

    def body(i, _):
        ref[pl.ds(i * 16, 16)] = v
        return 0

    lax.fori_loop(0, nwords // 16, body, 0)


def _fill_f32_2d(ref, nrows, ncols, value):
    v = jnp.full((16,), value, jnp.float32)

    def body(i, _):
        for k in range(ncols // 16):
            ref[i, pl.ds(k * 16, 16)] = v
        return 0

    lax.fori_loop(0, nrows, body, 0)


@functools.partial(
    pl.kernel,
    out_type=(
        jax.ShapeDtypeStruct((2 * NP, H), jnp.float32),
        jax.ShapeDtypeStruct((2 * NP, H), jnp.float32),
        jax.ShapeDtypeStruct((2, NP), jnp.float32),
    ),
    mesh=plsc.VectorSubcoreMesh(core_axis_name="c", subcore_axis_name="s"),
    scratch_types=[
        pltpu.VMEM((CHUNKS, B), jnp.int32),
        pltpu.VMEM((CHUNKS, B), jnp.int32),
        pltpu.VMEM((CHUNKS, B), jnp.int32),
        pltpu.VMEM((B, H), jnp.float32),
        pltpu.VMEM((B, H), jnp.float32),
        pltpu.VMEM((B,), jnp.float32),
        pltpu.VMEM((ZR, H), jnp.float32),
        pltpu.VMEM((NP,), jnp.float32),
        pltpu.VMEM_SHARED((NP, H), jnp.float32),
        pltpu.VMEM_SHARED((NP,), jnp.float32),
        pltpu.SemaphoreType.DMA,
        pltpu.SemaphoreType.DMA,
    ],
    compiler_params=pltpu.CompilerParams(use_tc_tiling_on_sc=False),
)
def _edge_agg(hw_hbm, rw_hbm, srca_hbm, srcb_hbm, eta_hbm, etb_hbm, dst_hbm,
              agg_a_out, agg_b_out, deg_out,
              src_v, dst_v, et_v, rows, rrows, ones, stg, dstg, agg_sh,
              deg_sh, sem1, sem2):
    cid = lax.axis_index("c")
    sid = lax.axis_index("s")
    wid = sid * NC + cid
    base = sid * RPT

    _fill_f32_1d(ones, B, 1.0)
    pltpu.sync_copy(dst_hbm.at[wid], dst_v)

    @pl.when(sid == 0)
    def _():
        _fill_f32_1d(dstg, NP, 0.0)
        pltpu.sync_copy(dstg, deg_sh)

    def one_pass(src_hbm, et_hbm, agg_out, with_deg):
        _fill_f32_2d(stg, ZR, H, 0.0)
        for k in range(RPT // ZR):
            pltpu.sync_copy(stg, agg_sh.at[pl.ds(base + k * ZR, ZR)])
        pltpu.sync_copy(src_hbm.at[wid], src_v)
        pltpu.sync_copy(et_hbm.at[wid], et_v)
        plsc.subcore_barrier()

        def body(j, _):
            cp1 = pltpu.async_copy(hw_hbm.at[src_v.at[j]], rows, sem1)
            cp2 = pltpu.async_copy(rw_hbm.at[et_v.at[j]], rrows, sem2)
            cp1.wait()
            cp2.wait()
            pltpu.sync_copy(rows, agg_sh.at[dst_v.at[j]], add=True)
            pltpu.sync_copy(rrows, agg_sh.at[dst_v.at[j]], add=True)
            if with_deg:
                pltpu.sync_copy(ones, deg_sh.at[dst_v.at[j]], add=True)
            return 0

        lax.fori_loop(0, CHUNKS, body, 0)
        plsc.subcore_barrier()

        for k in range(RPT // ZR):
            r0 = base + k * ZR
            pltpu.sync_copy(agg_sh.at[pl.ds(r0, ZR)], stg)
            pltpu.sync_copy(stg, agg_out.at[pl.ds(cid * NP + r0, ZR)])
        plsc.subcore_barrier()

    one_pass(srca_hbm, eta_hbm, agg_a_out, True)
    one_pass(srcb_hbm, etb_hbm, agg_b_out, False)

    @pl.when(sid == 0)
    def _():
        pltpu.sync_copy(deg_sh, dstg)
        pltpu.sync_copy(dstg, deg_out.at[cid])


def _mm2_kernel(x_ref, wa_ref, wb_ref, oa_ref, ob_ref):
    x = x_ref[...]
    oa_ref[...] = jnp.dot(x, wa_ref[...], preferred_element_type=jnp.float32)
    ob_ref[...] = jnp.dot(x, wb_ref[...], preferred_element_type=jnp.float32)


def _mm2(x, wa, wb, br):
    n = x.shape[0]
    grid = n // br
    return pl.pallas_call(
        _mm2_kernel,
        grid=(grid,),
        in_specs=[
            pl.BlockSpec((br, D), lambda i: (i, 0)),
            pl.BlockSpec((D, D), lambda i: (0, 0)),
            pl.BlockSpec((D, D), lambda i: (0, 0)),
        ],
        out_specs=[
            pl.BlockSpec((br, D), lambda i: (i, 0)),
            pl.BlockSpec((br, D), lambda i: (i, 0)),
        ],
        out_shape=[
            jax.ShapeDtypeStruct((n, D), jnp.float32),
            jax.ShapeDtypeStruct((n, D), jnp.float32),
        ],
    )(x, wa, wb)


def _rrelu(x):
    return jnp.where(x >= 0, x, RRELU_SLOPE * x)


def _agg_specs(br, nblk):
    half = pl.BlockSpec((br, H), lambda i: (i, 0))
    halfp = pl.BlockSpec((br, H), lambda i, n=nblk: (i + n, 0))
    one = pl.BlockSpec((br, 1), lambda i: (i, 0))
    onep = pl.BlockSpec((br, 1), lambda i, n=nblk: (i + n, 0))
    return [half, halfp, half, halfp, one, onep]


def _halves(p0a_ref, p1a_ref, p0b_ref, p1b_ref, d0_ref, d1_ref, hla_ref,
            hlb_ref):
    norm = 1.0 / jnp.maximum(d0_ref[...] + d1_ref[...], 1.0)
    ha = _rrelu((p0a_ref[...] + p1a_ref[...]) * norm + hla_ref[...])
    hb = _rrelu((p0b_ref[...] + p1b_ref[...]) * norm + hlb_ref[...])
    return ha, hb


def _combine_mm_kernel(p0a_ref, p1a_ref, p0b_ref, p1b_ref, d0_ref, d1_ref,
                       hla_ref, hlb_ref, wa_ref, wb_ref, oa_ref, ob_ref):
    ha, hb = _halves(p0a_ref, p1a_ref, p0b_ref, p1b_ref, d0_ref, d1_ref,
                     hla_ref, hlb_ref)
    wa = wa_ref[...]
    wb = wb_ref[...]
    oa_ref[...] = (jnp.dot(ha, wa[:H], preferred_element_type=jnp.float32)
                   + jnp.dot(hb, wa[H:], preferred_element_type=jnp.float32))
    ob_ref[...] = (jnp.dot(ha, wb[:H], preferred_element_type=jnp.float32)
                   + jnp.dot(hb, wb[H:], preferred_element_type=jnp.float32))


def _combine_mm(agg_a, agg_b, deg2, hla, hlb, wa, wb, br):
    grid = NP // br
    row = pl.BlockSpec((br, D), lambda i: (i, 0))
    half = pl.BlockSpec((br, H), lambda i: (i, 0))
    wspec = pl.BlockSpec((D, D), lambda i: (0, 0))
    return pl.pallas_call(
        _combine_mm_kernel,
        grid=(grid,),
        in_specs=_agg_specs(br, grid) + [half, half, wspec, wspec],
        out_specs=[row, row],
        out_shape=[
            jax.ShapeDtypeStruct((NP, D), jnp.float32),
            jax.ShapeDtypeStruct((NP, D), jnp.float32),
        ],
    )(agg_a, agg_a, agg_b, agg_b, deg2, deg2, hla, hlb, wa, wb)


def _combine_final_kernel(p0a_ref, p1a_ref, p0b_ref, p1b_ref, d0_ref, d1_ref,
                          hla_ref, hlb_ref, oa_ref, ob_ref):
    ha, hb = _halves(p0a_ref, p1a_ref, p0b_ref, p1b_ref, d0_ref, d1_ref,
                     hla_ref, hlb_ref)
    oa_ref[...] = ha
    ob_ref[...] = hb


def _combine_final(agg_a, agg_b, deg2, hla, hlb, br):
    grid = NP // br
    half = pl.BlockSpec((br, H), lambda i: (i, 0))
    return pl.pallas_call(
        _combine_final_kernel,
        grid=(grid,),
        in_specs=_agg_specs(br, grid) + [half, half],
        out_specs=[half, half],
        out_shape=[
            jax.ShapeDtypeStruct((NP, H), jnp.float32),
            jax.ShapeDtypeStruct((NP, H), jnp.float32),
        ],
    )(agg_a, agg_a, agg_b, agg_b, deg2, deg2, hla, hlb)


def kernel(emb_ent, emb_rel, Wn0, Wl0, Wn1, Wl1, edge_index, edge_type):
    src = edge_index[0]
    dst = edge_index[1].reshape(NW, CHUNKS, B)
    et = edge_type
    srca = (src * 2).reshape(NW, CHUNKS, B)
    srcb = (src * 2 + 1).reshape(NW, CHUNKS, B)
    eta = (et * 2).reshape(NW, CHUNKS, B)
    etb = (et * 2 + 1).reshape(NW, CHUNKS, B)

    emb_p = jnp.pad(emb_ent, ((0, NP - N), (0, 0)))
    hw0, hl0 = _mm2(emb_p, Wn0, Wl0, br=1280)
    rel_p = jnp.pad(emb_rel, ((0, 512 - NREL), (0, 0)))
    rw0, rw1 = _mm2(rel_p, Wn0, Wn1, br=512)

    agg0a, agg0b, deg = _edge_agg(hw0.reshape(2 * NP, H),
                                  rw0.reshape(1024, H),
                                  srca, srcb, eta, etb, dst)
    deg2 = deg.reshape(2 * NP, 1)

    hla0, hlb0 = hl0[:, :H], hl0[:, H:]
    hw1, hl1 = _combine_mm(agg0a, agg0b, deg2, hla0, hlb0, Wn1, Wl1, br=1280)

    agg1a, agg1b, _ = _edge_agg(hw1.reshape(2 * NP, H),
                                rw1.reshape(1024, H),
                                srca, srcb, eta, etb, dst)

    ha, hb = _combine_final(agg1a, agg1b, deg2, hl1[:, :H], hl1[:, H:],
                            br=1280)
    return jnp.concatenate([ha[:N], hb[:N]], axis=1)


# speedup vs baseline: 6.7598x; 1.4004x over previous
import functools

import jax
import jax.numpy as jnp
from jax import lax
from jax.experimental import pallas as pl
from jax.experimental.pallas import tpu as pltpu
from jax.experimental.pallas import tpu_sc as plsc

N = 10000
E = 320000
D = 128
H = D // 2
NREL = 460
RRELU_SLOPE = (1.0 / 8.0 + 1.0 / 3.0) / 2.0

NP = 10240
NC = 2
NS = 16
NW = NC * NS
EW = E // NW
B = 80
CHUNKS = EW // B
RPT = NP // NS
ZR = 128


def _fill_f32_1d(ref, nwords, value):
    v = jnp.full((16,), value, jnp.float32)

    def body(i, _):
        ref[pl.ds(i * 16, 16)] = v
        return 0

    lax.fori_loop(0, nwords // 16, body, 0)


def _fill_f32_2d(ref, nrows, ncols, value):
    v = jnp.full((16,), value, jnp.float32)

    def body(i, _):
        for k in range(ncols // 16):
            ref[i, pl.ds(k * 16, 16)] = v
        return 0

    lax.fori_loop(0, nrows, body, 0)


@functools.partial(
    pl.kernel,
    out_type=(
        jax.ShapeDtypeStruct((2 * NP, H), jnp.float32),
        jax.ShapeDtypeStruct((2 * NP, H), jnp.float32),
        jax.ShapeDtypeStruct((2, NP), jnp.float32),
    ),
    mesh=plsc.VectorSubcoreMesh(core_axis_name="c", subcore_axis_name="s"),
    scratch_types=[
        pltpu.VMEM((CHUNKS, B), jnp.int32),
        pltpu.VMEM((CHUNKS, B), jnp.int32),
        pltpu.VMEM((CHUNKS, B), jnp.int32),
        pltpu.VMEM((B, H), jnp.float32),
        pltpu.VMEM((B, H), jnp.float32),
        pltpu.VMEM((B, H), jnp.float32),
        pltpu.VMEM((B, H), jnp.float32),
        pltpu.VMEM((B,), jnp.float32),
        pltpu.VMEM((ZR, H), jnp.float32),
        pltpu.VMEM((NP,), jnp.float32),
        pltpu.VMEM_SHARED((NP, H), jnp.float32),
        pltpu.VMEM_SHARED((NP,), jnp.float32),
        pltpu.VMEM_SHARED((1024, H), jnp.float32),
        pltpu.SemaphoreType.DMA,
        pltpu.SemaphoreType.DMA,
        pltpu.SemaphoreType.DMA,
        pltpu.SemaphoreType.DMA,
    ],
    compiler_params=pltpu.CompilerParams(use_tc_tiling_on_sc=False),
)
def _edge_agg(hw_hbm, rw_hbm, srca_hbm, srcb_hbm, eta_hbm, etb_hbm, dst_hbm,
              agg_a_out, agg_b_out, deg_out,
              src_v, dst_v, et_v, rows0, rows1, rrows0, rrows1, ones, stg,
              dstg, agg_sh, deg_sh, rw_sh, sh0, sr0, sh1, sr1):
    cid = lax.axis_index("c")
    sid = lax.axis_index("s")
    wid = sid * NC + cid
    base = sid * RPT

    _fill_f32_1d(ones, B, 1.0)
    pltpu.sync_copy(dst_hbm.at[wid], dst_v)

    pltpu.sync_copy(rw_hbm.at[pl.ds(sid * 64, 64)], stg.at[pl.ds(0, 64)])
    pltpu.sync_copy(stg.at[pl.ds(0, 64)], rw_sh.at[pl.ds(sid * 64, 64)])

    @pl.when(sid == 0)
    def _():
        _fill_f32_1d(dstg, NP, 0.0)
        pltpu.sync_copy(dstg, deg_sh)

    def one_pass(src_hbm, et_hbm, agg_out, with_deg):
        _fill_f32_2d(stg, ZR, H, 0.0)
        for k in range(RPT // ZR):
            pltpu.sync_copy(stg, agg_sh.at[pl.ds(base + k * ZR, ZR)])
        pltpu.sync_copy(src_hbm.at[wid], src_v)
        pltpu.sync_copy(et_hbm.at[wid], et_v)
        plsc.subcore_barrier()

        bufs = ((rows0, rrows0, sh0, sr0), (rows1, rrows1, sh1, sr1))

        def start(j, buf):
            rows, rrows, sh, sr = buf
            pltpu.async_copy(hw_hbm.at[src_v.at[j]], rows, sh)
            pltpu.async_copy(rw_sh.at[et_v.at[j]], rrows, sr)

        def finish(j, buf):
            rows, rrows, sh, sr = buf
            pltpu.make_async_copy(hw_hbm.at[src_v.at[j]], rows, sh).wait()
            pltpu.make_async_copy(rw_sh.at[et_v.at[j]], rrows, sr).wait()
            pltpu.sync_copy(rows, agg_sh.at[dst_v.at[j]], add=True)
            pltpu.sync_copy(rrows, agg_sh.at[dst_v.at[j]], add=True)
            if with_deg:
                pltpu.sync_copy(ones, deg_sh.at[dst_v.at[j]], add=True)

        start(0, bufs[0])

        def body(i, _):
            j0 = 2 * i
            for (j, buf, nbuf) in ((j0, bufs[0], bufs[1]),
                                   (j0 + 1, bufs[1], bufs[0])):
                @pl.when(j + 1 < CHUNKS)
                def _(j=j, nbuf=nbuf):
                    start(j + 1, nbuf)

                @pl.when(j < CHUNKS)
                def _(j=j, buf=buf):
                    finish(j, buf)
            return 0

        lax.fori_loop(0, (CHUNKS + 1) // 2, body, 0)
        plsc.subcore_barrier()

        for k in range(RPT // ZR):
            r0 = base + k * ZR
            pltpu.sync_copy(agg_sh.at[pl.ds(r0, ZR)], stg)
            pltpu.sync_copy(stg, agg_out.at[pl.ds(cid * NP + r0, ZR)])
        plsc.subcore_barrier()

    one_pass(srca_hbm, eta_hbm, agg_a_out, True)
    one_pass(srcb_hbm, etb_hbm, agg_b_out, False)

    @pl.when(sid == 0)
    def _():
        pltpu.sync_copy(deg_sh, dstg)
        pltpu.sync_copy(dstg, deg_out.at[cid])


def _mm2_kernel(x_ref, wa_ref, wb_ref, oa_ref, ob_ref):
    x = x_ref[...]
    oa_ref[...] = jnp.dot(x, wa_ref[...], preferred_element_type=jnp.float32)
    ob_ref[...] = jnp.dot(x, wb_ref[...], preferred_element_type=jnp.float32)


def _mm2(x, wa, wb, br):
    n = x.shape[0]
    grid = n // br
    return pl.pallas_call(
        _mm2_kernel,
        grid=(grid,),
        in_specs=[
            pl.BlockSpec((br, D), lambda i: (i, 0)),
            pl.BlockSpec((D, D), lambda i: (0, 0)),
            pl.BlockSpec((D, D), lambda i: (0, 0)),
        ],
        out_specs=[
            pl.BlockSpec((br, D), lambda i: (i, 0)),
            pl.BlockSpec((br, D), lambda i: (i, 0)),
        ],
        out_shape=[
            jax.ShapeDtypeStruct((n, D), jnp.float32),
            jax.ShapeDtypeStruct((n, D), jnp.float32),
        ],
    )(x, wa, wb)


def _rrelu(x):
    return jnp.where(x >= 0, x, RRELU_SLOPE * x)


def _agg_specs(br, nblk):
    half = pl.BlockSpec((br, H), lambda i: (i, 0))
    halfp = pl.BlockSpec((br, H), lambda i, n=nblk: (i + n, 0))
    one = pl.BlockSpec((br, 1), lambda i: (i, 0))
    onep = pl.BlockSpec((br, 1), lambda i, n=nblk: (i + n, 0))
    return [half, halfp, half, halfp, one, onep]


def _halves(p0a_ref, p1a_ref, p0b_ref, p1b_ref, d0_ref, d1_ref, hla_ref,
            hlb_ref):
    norm = 1.0 / jnp.maximum(d0_ref[...] + d1_ref[...], 1.0)
    ha = _rrelu((p0a_ref[...] + p1a_ref[...]) * norm + hla_ref[...])
    hb = _rrelu((p0b_ref[...] + p1b_ref[...]) * norm + hlb_ref[...])
    return ha, hb


def _combine_mm_kernel(p0a_ref, p1a_ref, p0b_ref, p1b_ref, d0_ref, d1_ref,
                       hla_ref, hlb_ref, wa_ref, wb_ref, oa_ref, ob_ref):
    ha, hb = _halves(p0a_ref, p1a_ref, p0b_ref, p1b_ref, d0_ref, d1_ref,
                     hla_ref, hlb_ref)
    wa = wa_ref[...]
    wb = wb_ref[...]
    oa_ref[...] = (jnp.dot(ha, wa[:H], preferred_element_type=jnp.float32)
                   + jnp.dot(hb, wa[H:], preferred_element_type=jnp.float32))
    ob_ref[...] = (jnp.dot(ha, wb[:H], preferred_element_type=jnp.float32)
                   + jnp.dot(hb, wb[H:], preferred_element_type=jnp.float32))


def _combine_mm(agg_a, agg_b, deg2, hla, hlb, wa, wb, br):
    grid = NP // br
    row = pl.BlockSpec((br, D), lambda i: (i, 0))
    half = pl.BlockSpec((br, H), lambda i: (i, 0))
    wspec = pl.BlockSpec((D, D), lambda i: (0, 0))
    return pl.pallas_call(
        _combine_mm_kernel,
        grid=(grid,),
        in_specs=_agg_specs(br, grid) + [half, half, wspec, wspec],
        out_specs=[row, row],
        out_shape=[
            jax.ShapeDtypeStruct((NP, D), jnp.float32),
            jax.ShapeDtypeStruct((NP, D), jnp.float32),
        ],
    )(agg_a, agg_a, agg_b, agg_b, deg2, deg2, hla, hlb, wa, wb)


def _combine_final_kernel(p0a_ref, p1a_ref, p0b_ref, p1b_ref, d0_ref, d1_ref,
                          hla_ref, hlb_ref, oa_ref, ob_ref):
    ha, hb = _halves(p0a_ref, p1a_ref, p0b_ref, p1b_ref, d0_ref, d1_ref,
                     hla_ref, hlb_ref)
    oa_ref[...] = ha
    ob_ref[...] = hb


def _combine_final(agg_a, agg_b, deg2, hla, hlb, br):
    grid = NP // br
    half = pl.BlockSpec((br, H), lambda i: (i, 0))
    return pl.pallas_call(
        _combine_final_kernel,
        grid=(grid,),
        in_specs=_agg_specs(br, grid) + [half, half],
        out_specs=[half, half],
        out_shape=[
            jax.ShapeDtypeStruct((NP, H), jnp.float32),
            jax.ShapeDtypeStruct((NP, H), jnp.float32),
        ],
    )(agg_a, agg_a, agg_b, agg_b, deg2, deg2, hla, hlb)


def kernel(emb_ent, emb_rel, Wn0, Wl0, Wn1, Wl1, edge_index, edge_type):
    src = edge_index[0]
    dst = edge_index[1].reshape(NW, CHUNKS, B)
    et = edge_type
    srca = (src * 2).reshape(NW, CHUNKS, B)
    srcb = (src * 2 + 1).reshape(NW, CHUNKS, B)
    eta = (et * 2).reshape(NW, CHUNKS, B)
    etb = (et * 2 + 1).reshape(NW, CHUNKS, B)

    emb_p = jnp.pad(emb_ent, ((0, NP - N), (0, 0)))
    hw0, hl0 = _mm2(emb_p, Wn0, Wl0, br=1280)
    rel_p = jnp.pad(emb_rel, ((0, 512 - NREL), (0, 0)))
    rw0, rw1 = _mm2(rel_p, Wn0, Wn1, br=512)

    agg0a, agg0b, deg = _edge_agg(hw0.reshape(2 * NP, H),
                                  rw0.reshape(1024, H),
                                  srca, srcb, eta, etb, dst)
    deg2 = deg.reshape(2 * NP, 1)

    hla0, hlb0 = hl0[:, :H], hl0[:, H:]
    hw1, hl1 = _combine_mm(agg0a, agg0b, deg2, hla0, hlb0, Wn1, Wl1, br=1280)

    agg1a, agg1b, _ = _edge_agg(hw1.reshape(2 * NP, H),
                                rw1.reshape(1024, H),
                                srca, srcb, eta, etb, dst)

    ha, hb = _combine_final(agg1a, agg1b, deg2, hl1[:, :H], hl1[:, H:],
                            br=1280)
    return jnp.concatenate([ha[:N], hb[:N]], axis=1)
